# baseline (speedup 1.0000x reference)
import functools

import jax
import jax.numpy as jnp
from jax.experimental import pallas as pl
from jax.experimental.pallas import tpu as pltpu

B, N = 4, 20000
NC, NM = 80, 32
PH = PW = 160
MAX_OBJ = 100
IOU_THR = 0.45
SCORE_THR = 0.25
RES = 56
SCALE = 0.25

BN = 2048
NB = (N + BN - 1) // BN
NP = NB * BN


def _nms_body(xt_ref, nd_ref, bx1_ref, by1_ref, bx2_ref, by2_ref,
              dsc_ref, dcls_ref, dcoef_ref,
              sc_s, x1_s, y1_s, x2_s, y2_s, ar_s, cid_s, cf_s,
              act_s, didx_s):
    i = pl.program_id(0)
    blk = xt_ref[0]

    cx = blk[:, :, 0:1]
    cy = blk[:, :, 1:2]
    w = blk[:, :, 2:3]
    h = blk[:, :, 3:4]
    obj = blk[:, :, 4:5]
    cls = blk[:, :, 5:5 + NC]
    cmax = jnp.max(cls, axis=2, keepdims=True)
    cio = jax.lax.broadcasted_iota(jnp.int32, (B, BN, NC), 2)
    cid = jnp.min(jnp.where(cls == cmax, cio, NC), axis=2, keepdims=True)

    x1 = cx - w / 2
    y1 = cy - h / 2
    x2 = cx + w / 2
    y2 = cy + h / 2
    cols = jnp.concatenate(
        [x1, y1, x2, y2, obj * cmax, (x2 - x1) * (y2 - y1),
         cid.astype(jnp.float32), jnp.zeros_like(cx),
         blk[:, :, 5 + NC:5 + NC + NM]], axis=2)
    colsT = jnp.transpose(cols, (0, 2, 1))
    sl = pl.ds(i * BN, BN)
    x1_s[:, sl] = colsT[:, 0, :]
    y1_s[:, sl] = colsT[:, 1, :]
    x2_s[:, sl] = colsT[:, 2, :]
    y2_s[:, sl] = colsT[:, 3, :]
    sc_s[:, sl] = colsT[:, 4, :]
    ar_s[:, sl] = colsT[:, 5, :]
    cid_s[:, sl] = colsT[:, 6, :].astype(jnp.int32)
    cf_s[:, :, sl] = colsT[:, 8:8 + NM, :]

    @pl.when(i == NB - 1)
    def _run_nms():
        scores = sc_s[...]
        nio = jax.lax.broadcasted_iota(jnp.int32, (B, NP), 1)
        act_s[...] = jnp.where((scores > SCORE_THR) & (nio < N), 1.0, 0.0)
        bx1 = x1_s[...]
        by1 = y1_s[...]
        bx2 = x2_s[...]
        by2 = y2_s[...]
        areas = ar_s[...]
        cidv = cid_s[...]
        neg = jnp.float32(-jnp.inf)

        tio = jax.lax.broadcasted_iota(jnp.int32, (B, MAX_OBJ), 1)

        def step(t, carry):
            cnt, di, ox1, oy1, ox2, oy2, osc, ocl = carry
            act = act_s[...]
            s = jnp.where(act > 0, scores, neg)
            m = jnp.max(s, axis=1, keepdims=True)
            idx = jnp.min(jnp.where(s == m, nio, NP), axis=1, keepdims=True)
            oh = nio == idx
            valid = m > SCORE_THR

            px1 = jnp.sum(jnp.where(oh, bx1, 0.0), axis=1, keepdims=True)
            py1 = jnp.sum(jnp.where(oh, by1, 0.0), axis=1, keepdims=True)
            px2 = jnp.sum(jnp.where(oh, bx2, 0.0), axis=1, keepdims=True)
            py2 = jnp.sum(jnp.where(oh, by2, 0.0), axis=1, keepdims=True)
            pcls = jnp.sum(jnp.where(oh, cidv, 0), axis=1, keepdims=True)

            xx1 = jnp.maximum(px1, bx1)
            yy1 = jnp.maximum(py1, by1)
            xx2 = jnp.minimum(px2, bx2)
            yy2 = jnp.minimum(py2, by2)
            inter = (jnp.clip(xx2 - xx1, 0.0)
                     * jnp.clip(yy2 - yy1, 0.0))
            a1 = (px2 - px1) * (py2 - py1)
            iou = inter / (a1 + areas - inter + 1e-9)
            keep = (act > 0) & (iou <= IOU_THR) & jnp.logical_not(oh)
            act_s[...] = jnp.where(keep, 1.0, 0.0)

            here = tio == t
            di = jnp.where(here, idx, di)
            ox1 = jnp.where(here, px1, ox1)
            oy1 = jnp.where(here, py1, oy1)
            ox2 = jnp.where(here, px2, ox2)
            oy2 = jnp.where(here, py2, oy2)
            osc = jnp.where(here, jnp.where(valid, m, 0.0), osc)
            ocl = jnp.where(here, pcls, ocl)
            return (cnt + valid[:, 0].astype(jnp.float32),
                    di, ox1, oy1, ox2, oy2, osc, ocl)

        zf = jnp.zeros((B, MAX_OBJ), jnp.float32)
        zi = jnp.zeros((B, MAX_OBJ), jnp.int32)
        carry0 = (jnp.zeros((B,), jnp.float32), zi, zf, zf, zf, zf, zf, zi)
        cnt, di, ox1, oy1, ox2, oy2, osc, ocl = jax.lax.fori_loop(
            0, MAX_OBJ, step, carry0)
        nd_ref[...] = cnt.astype(jnp.int32)[:, None]
        didx_s[...] = di
        bx1_ref[...] = ox1
        by1_ref[...] = oy1
        bx2_ref[...] = ox2
        by2_ref[...] = oy2
        dsc_ref[...] = osc
        dcls_ref[...] = ocl

        CHN = 4096
        for b in range(B):
            idxb = didx_s[b, :][:, None]
            acc = jnp.zeros((MAX_OBJ, NM), jnp.float32)
            for c0 in range(0, NP, CHN):
                cio2 = (jax.lax.broadcasted_iota(
                    jnp.int32, (MAX_OBJ, CHN), 1) + c0)
                ohc = (cio2 == idxb).astype(jnp.float32)
                cfc = cf_s[b, :, pl.ds(c0, CHN)]
                acc = acc + jax.lax.dot_general(
                    ohc, cfc, (((1,), (1,)), ((), ())),
                    preferred_element_type=jnp.float32)
            dcoef_ref[b] = acc


def _mask_body(proto_ref, bx1_ref, by1_ref, bx2_ref, by2_ref, dcoef_ref,
               out_ref, mall_s, sy_s, sxt_s):
    coefs = dcoef_ref[0]
    res = jnp.dot(coefs, proto_ref[0],
                  preferred_element_type=jnp.float32)
    mall_s[...] = res.reshape(MAX_OBJ, PH, PW)

    def roi_axis(lo_ref, hi_ref, size):
        lo = lo_ref[0, 0].reshape(MAX_OBJ, 1) * SCALE - 0.5
        hi = hi_ref[0, 0].reshape(MAX_OBJ, 1) * SCALE - 0.5
        g = (jax.lax.broadcasted_iota(jnp.int32, (1, RES), 1)
             .astype(jnp.float32) + 0.5) / RES
        gc = lo + g * (hi - lo)
        return jnp.clip(gc, 0.0, size - 1.0)

    gx = roi_axis(bx1_ref, bx2_ref, PW)
    gy = roi_axis(by1_ref, by2_ref, PH)
    ky = jax.lax.broadcasted_iota(
        jnp.int32, (MAX_OBJ, RES, PH), 2).astype(jnp.float32)
    sy_s[...] = jnp.maximum(1.0 - jnp.abs(gy[:, :, None] - ky), 0.0)
    kx = jax.lax.broadcasted_iota(
        jnp.int32, (MAX_OBJ, PW, RES), 1).astype(jnp.float32)
    sxt_s[...] = jnp.maximum(1.0 - jnp.abs(kx - gx[:, None, :]), 0.0)

    def roi(r, _):
        mm = mall_s[r]
        t = jnp.dot(sy_s[r], mm, preferred_element_type=jnp.float32)
        o = jnp.dot(t, sxt_s[r], preferred_element_type=jnp.float32)
        out_ref[0, r] = jax.nn.sigmoid(o)
        return 0

    jax.lax.fori_loop(0, MAX_OBJ, roi, 0)


@jax.jit
def kernel(x0, proto):
    F = x0.shape[-1]
    protof = proto.reshape(B, NM, PH * PW)

    outs = pl.pallas_call(
        _nms_body,
        grid=(NB,),
        in_specs=[pl.BlockSpec((1, B, BN, F), lambda i: (0, 0, i, 0))],
        out_specs=[
            pl.BlockSpec((B, 1), lambda i: (0, 0)),
            pl.BlockSpec((B, MAX_OBJ), lambda i: (0, 0)),
            pl.BlockSpec((B, MAX_OBJ), lambda i: (0, 0)),
            pl.BlockSpec((B, MAX_OBJ), lambda i: (0, 0)),
            pl.BlockSpec((B, MAX_OBJ), lambda i: (0, 0)),
            pl.BlockSpec((B, MAX_OBJ), lambda i: (0, 0)),
            pl.BlockSpec((B, MAX_OBJ), lambda i: (0, 0)),
            pl.BlockSpec((B, MAX_OBJ, NM), lambda i: (0, 0, 0)),
        ],
        out_shape=[
            jax.ShapeDtypeStruct((B, 1), jnp.int32),
            jax.ShapeDtypeStruct((B, MAX_OBJ), jnp.float32),
            jax.ShapeDtypeStruct((B, MAX_OBJ), jnp.float32),
            jax.ShapeDtypeStruct((B, MAX_OBJ), jnp.float32),
            jax.ShapeDtypeStruct((B, MAX_OBJ), jnp.float32),
            jax.ShapeDtypeStruct((B, MAX_OBJ), jnp.float32),
            jax.ShapeDtypeStruct((B, MAX_OBJ), jnp.int32),
            jax.ShapeDtypeStruct((B, MAX_OBJ, NM), jnp.float32),
        ],
        scratch_shapes=[
            pltpu.VMEM((B, NP), jnp.float32),
            pltpu.VMEM((B, NP), jnp.float32),
            pltpu.VMEM((B, NP), jnp.float32),
            pltpu.VMEM((B, NP), jnp.float32),
            pltpu.VMEM((B, NP), jnp.float32),
            pltpu.VMEM((B, NP), jnp.float32),
            pltpu.VMEM((B, NP), jnp.int32),
            pltpu.VMEM((B, NM, NP), jnp.float32),
            pltpu.VMEM((B, NP), jnp.float32),
            pltpu.VMEM((B, MAX_OBJ), jnp.int32),
        ],
    )(x0[None])
    num_det, dbx1, dby1, dbx2, dby2, dsc, dcls, dcoef = outs

    masks4 = pl.pallas_call(
        _mask_body,
        grid=(B,),
        in_specs=[
            pl.BlockSpec((1, NM, PH * PW), lambda b: (b, 0, 0)),
            pl.BlockSpec((1, 1, MAX_OBJ), lambda b: (b, 0, 0)),
            pl.BlockSpec((1, 1, MAX_OBJ), lambda b: (b, 0, 0)),
            pl.BlockSpec((1, 1, MAX_OBJ), lambda b: (b, 0, 0)),
            pl.BlockSpec((1, 1, MAX_OBJ), lambda b: (b, 0, 0)),
            pl.BlockSpec((1, MAX_OBJ, NM), lambda b: (b, 0, 0)),
        ],
        out_specs=pl.BlockSpec((1, MAX_OBJ, RES, RES), lambda b: (b, 0, 0, 0)),
        out_shape=jax.ShapeDtypeStruct((B, MAX_OBJ, RES, RES), jnp.float32),
        scratch_shapes=[
            pltpu.VMEM((MAX_OBJ, PH, PW), jnp.float32),
            pltpu.VMEM((MAX_OBJ, RES, PH), jnp.float32),
            pltpu.VMEM((MAX_OBJ, PW, RES), jnp.float32),
        ],
    )(protof, dbx1[:, None], dby1[:, None], dbx2[:, None], dby2[:, None],
      dcoef)

    det_boxes = jnp.stack([dbx1, dby1, dbx2, dby2], axis=-1)
    masks = masks4.reshape(B, MAX_OBJ, RES * RES)
    return num_det, det_boxes, dsc, dcls, masks


# speedup vs baseline: 1.2471x; 1.2471x over previous
import functools

import jax
import jax.numpy as jnp
from jax.experimental import pallas as pl
from jax.experimental.pallas import tpu as pltpu

B, N = 4, 20000
NC, NM = 80, 32
PH = PW = 160
MAX_OBJ = 100
IOU_THR = 0.45
SCORE_THR = 0.25
RES = 56
SCALE = 0.25

BN = 2048
NB = (N + BN - 1) // BN
NP = NB * BN


def _nms_body(xt_ref, nd_ref, bx1_ref, by1_ref, bx2_ref, by2_ref,
              dsc_ref, dcls_ref, dcoef_ref,
              sc_s, x1_s, y1_s, x2_s, y2_s, ar_s, cid_s, cf_s,
              act_s, didx_s):
    i = pl.program_id(0)
    blk = xt_ref[...]

    cx = blk[:, :, 0:1]
    cy = blk[:, :, 1:2]
    w = blk[:, :, 2:3]
    h = blk[:, :, 3:4]
    obj = blk[:, :, 4:5]
    cls = blk[:, :, 5:5 + NC]
    cmax = jnp.max(cls, axis=2, keepdims=True)
    cio = jax.lax.broadcasted_iota(jnp.int32, (B, BN, NC), 2)
    cid = jnp.min(jnp.where(cls == cmax, cio, NC), axis=2, keepdims=True)

    x1 = cx - w / 2
    y1 = cy - h / 2
    x2 = cx + w / 2
    y2 = cy + h / 2
    cols = jnp.concatenate(
        [x1, y1, x2, y2, obj * cmax, (x2 - x1) * (y2 - y1),
         cid.astype(jnp.float32), jnp.zeros_like(cx),
         blk[:, :, 5 + NC:5 + NC + NM]], axis=2)
    colsT = jnp.transpose(cols, (0, 2, 1))
    sl = pl.ds(i * BN, BN)
    x1_s[:, sl] = colsT[:, 0, :]
    y1_s[:, sl] = colsT[:, 1, :]
    x2_s[:, sl] = colsT[:, 2, :]
    y2_s[:, sl] = colsT[:, 3, :]
    sc_s[:, sl] = colsT[:, 4, :]
    ar_s[:, sl] = colsT[:, 5, :]
    cid_s[:, sl] = colsT[:, 6, :].astype(jnp.int32)
    cf_s[:, :, sl] = colsT[:, 8:8 + NM, :]

    @pl.when(i == NB - 1)
    def _run_nms():
        scores = sc_s[...]
        nio = jax.lax.broadcasted_iota(jnp.int32, (B, NP), 1)
        act_s[...] = jnp.where((scores > SCORE_THR) & (nio < N), 1.0, 0.0)
        bx1 = x1_s[...]
        by1 = y1_s[...]
        bx2 = x2_s[...]
        by2 = y2_s[...]
        areas = ar_s[...]
        cidv = cid_s[...]
        neg = jnp.float32(-jnp.inf)

        tio = jax.lax.broadcasted_iota(jnp.int32, (B, MAX_OBJ), 1)

        def step(t, carry):
            cnt, di, ox1, oy1, ox2, oy2, osc, ocl = carry
            act = act_s[...]
            s = jnp.where(act > 0, scores, neg)
            m = jnp.max(s, axis=1, keepdims=True)
            idx = jnp.min(jnp.where(s == m, nio, NP), axis=1, keepdims=True)
            oh = nio == idx
            valid = m > SCORE_THR

            px1 = jnp.sum(jnp.where(oh, bx1, 0.0), axis=1, keepdims=True)
            py1 = jnp.sum(jnp.where(oh, by1, 0.0), axis=1, keepdims=True)
            px2 = jnp.sum(jnp.where(oh, bx2, 0.0), axis=1, keepdims=True)
            py2 = jnp.sum(jnp.where(oh, by2, 0.0), axis=1, keepdims=True)
            pcls = jnp.sum(jnp.where(oh, cidv, 0), axis=1, keepdims=True)

            xx1 = jnp.maximum(px1, bx1)
            yy1 = jnp.maximum(py1, by1)
            xx2 = jnp.minimum(px2, bx2)
            yy2 = jnp.minimum(py2, by2)
            inter = (jnp.clip(xx2 - xx1, 0.0)
                     * jnp.clip(yy2 - yy1, 0.0))
            a1 = (px2 - px1) * (py2 - py1)
            iou = inter / (a1 + areas - inter + 1e-9)
            keep = (act > 0) & (iou <= IOU_THR) & jnp.logical_not(oh)
            act_s[...] = jnp.where(keep, 1.0, 0.0)

            here = tio == t
            di = jnp.where(here, idx, di)
            ox1 = jnp.where(here, px1, ox1)
            oy1 = jnp.where(here, py1, oy1)
            ox2 = jnp.where(here, px2, ox2)
            oy2 = jnp.where(here, py2, oy2)
            osc = jnp.where(here, jnp.where(valid, m, 0.0), osc)
            ocl = jnp.where(here, pcls, ocl)
            return (cnt + valid[:, 0].astype(jnp.float32),
                    di, ox1, oy1, ox2, oy2, osc, ocl)

        zf = jnp.zeros((B, MAX_OBJ), jnp.float32)
        zi = jnp.zeros((B, MAX_OBJ), jnp.int32)
        carry0 = (jnp.zeros((B,), jnp.float32), zi, zf, zf, zf, zf, zf, zi)
        cnt, di, ox1, oy1, ox2, oy2, osc, ocl = jax.lax.fori_loop(
            0, MAX_OBJ, step, carry0)
        nd_ref[...] = cnt.astype(jnp.int32)[:, None]
        didx_s[...] = di
        bx1_ref[...] = ox1
        by1_ref[...] = oy1
        bx2_ref[...] = ox2
        by2_ref[...] = oy2
        dsc_ref[...] = osc
        dcls_ref[...] = ocl

        CHN = 4096
        for b in range(B):
            idxb = didx_s[b, :][:, None]
            acc = jnp.zeros((MAX_OBJ, NM), jnp.float32)
            for c0 in range(0, NP, CHN):
                cio2 = (jax.lax.broadcasted_iota(
                    jnp.int32, (MAX_OBJ, CHN), 1) + c0)
                ohc = (cio2 == idxb).astype(jnp.float32)
                cfc = cf_s[b, :, pl.ds(c0, CHN)]
                acc = acc + jax.lax.dot_general(
                    ohc, cfc, (((1,), (1,)), ((), ())),
                    preferred_element_type=jnp.float32)
            dcoef_ref[b] = acc


def _mask_body(proto_ref, bx1_ref, by1_ref, bx2_ref, by2_ref, dcoef_ref,
               out_ref, mall_s, sy_s, sxt_s):
    coefs = dcoef_ref[0]
    res = jnp.dot(coefs, proto_ref[0].reshape(NM, PH * PW),
                  preferred_element_type=jnp.float32)
    mall_s[...] = res.reshape(MAX_OBJ, PH, PW)

    def roi_axis(lo_ref, hi_ref, size):
        lo = lo_ref[0, 0].reshape(MAX_OBJ, 1) * SCALE - 0.5
        hi = hi_ref[0, 0].reshape(MAX_OBJ, 1) * SCALE - 0.5
        g = (jax.lax.broadcasted_iota(jnp.int32, (1, RES), 1)
             .astype(jnp.float32) + 0.5) / RES
        gc = lo + g * (hi - lo)
        return jnp.clip(gc, 0.0, size - 1.0)

    gx = roi_axis(bx1_ref, bx2_ref, PW)
    gy = roi_axis(by1_ref, by2_ref, PH)
    ky = jax.lax.broadcasted_iota(
        jnp.int32, (MAX_OBJ, RES, PH), 2).astype(jnp.float32)
    sy_s[...] = jnp.maximum(1.0 - jnp.abs(gy[:, :, None] - ky), 0.0)
    kx = jax.lax.broadcasted_iota(
        jnp.int32, (MAX_OBJ, PW, RES), 1).astype(jnp.float32)
    sxt_s[...] = jnp.maximum(1.0 - jnp.abs(kx - gx[:, None, :]), 0.0)

    def roi(r, _):
        mm = mall_s[r]
        t = jnp.dot(sy_s[r], mm, preferred_element_type=jnp.float32)
        o = jnp.dot(t, sxt_s[r], preferred_element_type=jnp.float32)
        out_ref[0, r] = jax.nn.sigmoid(o)
        return 0

    jax.lax.fori_loop(0, MAX_OBJ, roi, 0)


@jax.jit
def kernel(x0, proto):
    F = x0.shape[-1]

    outs = pl.pallas_call(
        _nms_body,
        grid=(NB,),
        in_specs=[pl.BlockSpec((B, BN, F), lambda i: (0, i, 0))],
        out_specs=[
            pl.BlockSpec((B, 1), lambda i: (0, 0)),
            pl.BlockSpec((B, MAX_OBJ), lambda i: (0, 0)),
            pl.BlockSpec((B, MAX_OBJ), lambda i: (0, 0)),
            pl.BlockSpec((B, MAX_OBJ), lambda i: (0, 0)),
            pl.BlockSpec((B, MAX_OBJ), lambda i: (0, 0)),
            pl.BlockSpec((B, MAX_OBJ), lambda i: (0, 0)),
            pl.BlockSpec((B, MAX_OBJ), lambda i: (0, 0)),
            pl.BlockSpec((B, MAX_OBJ, NM), lambda i: (0, 0, 0)),
        ],
        out_shape=[
            jax.ShapeDtypeStruct((B, 1), jnp.int32),
            jax.ShapeDtypeStruct((B, MAX_OBJ), jnp.float32),
            jax.ShapeDtypeStruct((B, MAX_OBJ), jnp.float32),
            jax.ShapeDtypeStruct((B, MAX_OBJ), jnp.float32),
            jax.ShapeDtypeStruct((B, MAX_OBJ), jnp.float32),
            jax.ShapeDtypeStruct((B, MAX_OBJ), jnp.float32),
            jax.ShapeDtypeStruct((B, MAX_OBJ), jnp.int32),
            jax.ShapeDtypeStruct((B, MAX_OBJ, NM), jnp.float32),
        ],
        scratch_shapes=[
            pltpu.VMEM((B, NP), jnp.float32),
            pltpu.VMEM((B, NP), jnp.float32),
            pltpu.VMEM((B, NP), jnp.float32),
            pltpu.VMEM((B, NP), jnp.float32),
            pltpu.VMEM((B, NP), jnp.float32),
            pltpu.VMEM((B, NP), jnp.float32),
            pltpu.VMEM((B, NP), jnp.int32),
            pltpu.VMEM((B, NM, NP), jnp.float32),
            pltpu.VMEM((B, NP), jnp.float32),
            pltpu.VMEM((B, MAX_OBJ), jnp.int32),
        ],
    )(x0)
    num_det, dbx1, dby1, dbx2, dby2, dsc, dcls, dcoef = outs

    masks4 = pl.pallas_call(
        _mask_body,
        grid=(B,),
        in_specs=[
            pl.BlockSpec((1, NM, PH, PW), lambda b: (b, 0, 0, 0)),
            pl.BlockSpec((1, 1, MAX_OBJ), lambda b: (b, 0, 0)),
            pl.BlockSpec((1, 1, MAX_OBJ), lambda b: (b, 0, 0)),
            pl.BlockSpec((1, 1, MAX_OBJ), lambda b: (b, 0, 0)),
            pl.BlockSpec((1, 1, MAX_OBJ), lambda b: (b, 0, 0)),
            pl.BlockSpec((1, MAX_OBJ, NM), lambda b: (b, 0, 0)),
        ],
        out_specs=pl.BlockSpec((1, MAX_OBJ, RES, RES), lambda b: (b, 0, 0, 0)),
        out_shape=jax.ShapeDtypeStruct((B, MAX_OBJ, RES, RES), jnp.float32),
        scratch_shapes=[
            pltpu.VMEM((MAX_OBJ, PH, PW), jnp.float32),
            pltpu.VMEM((MAX_OBJ, RES, PH), jnp.float32),
            pltpu.VMEM((MAX_OBJ, PW, RES), jnp.float32),
        ],
    )(proto, dbx1[:, None], dby1[:, None], dbx2[:, None], dby2[:, None],
      dcoef)

    det_boxes = jnp.stack([dbx1, dby1, dbx2, dby2], axis=-1)
    masks = masks4.reshape(B, MAX_OBJ, RES * RES)
    return num_det, det_boxes, dsc, dcls, masks
